# baseline (speedup 1.0000x reference)
import functools

import jax
import jax.numpy as jnp
from jax import lax
from jax.experimental import pallas as pl
from jax.experimental.pallas import tpu as pltpu
from jax.experimental.pallas import tpu_sc as plsc

N = 10000
T = 8
D = 128
H = 4
DQ = 32
E = 160000

NTILES = 16
NCORES = 2
CHUNK = 72
NCH = 140
EPT = NCH * CHUNK
EPAD = EPT * NTILES
NPT = 624
NREM = N - NTILES * NPT
TPC = T // NCORES
IWCH = ((0, 128), (128, 128), (256, 128), (384, 128), (512, 112))

TN = T * N


def _spmm_body(hw, srcT, dst3, wrep, out, accum, srcv, dstv, wrepv, rows,
               gsem0, gsem1, stsem, dsem, scsem):
    c = lax.axis_index("c")
    s = lax.axis_index("s")
    gsems = (gsem0, gsem1)

    for jj in range(TPC):
        t = c * TPC + jj

        nbase = pl.multiple_of(s * NPT, 8)
        pltpu.sync_copy(hw.at[pl.ds(pl.multiple_of(t * N, 8) + nbase, NPT)],
                        accum.at[pl.ds(nbase, NPT)])

        @pl.when(s == 0)
        def _init_rem():
            pltpu.sync_copy(
                hw.at[pl.ds(pl.multiple_of(t * N + NTILES * NPT, 8), NREM)],
                accum.at[pl.ds(NTILES * NPT, NREM)])

        plsc.subcore_barrier()

        for b in range(2):
            pltpu.async_copy(srcT.at[t, s, b], srcv.at[b], stsem)
            pltpu.async_copy(dst3.at[s, b], dstv.at[b], stsem)
            pltpu.async_copy(wrep.at[s, b], wrepv.at[b], stsem)
        for b in range(2):
            pltpu.make_async_copy(srcT.at[t, s, b], srcv.at[b], stsem).wait()
            pltpu.make_async_copy(dst3.at[s, b], dstv.at[b], stsem).wait()
            pltpu.make_async_copy(wrep.at[s, b], wrepv.at[b], stsem).wait()
            pltpu.async_copy(hw.at[srcv.at[b]], rows.at[b], gsems[b])

        @pl.loop(0, NCH, step=2)
        def _chunk2(ch):
            for b in range(2):
                cur = ch + b
                nxt = cur + 2
                pltpu.make_async_copy(hw.at[srcv.at[b]], rows.at[b],
                                      gsems[b]).wait()

                @pl.when(nxt < NCH)
                def _stage_src():
                    pltpu.async_copy(srcT.at[t, s, nxt], srcv.at[b], stsem)

                @pl.loop(0, CHUNK)
                def _edge(i):
                    wspl = wrepv[b, i]
                    for k in range(D // 16):
                        sl = pl.ds(k * 16, 16)
                        rows[b, i, sl] = rows[b, i, sl] * wspl

                @pl.when(nxt < NCH)
                def _stage_w():
                    pltpu.async_copy(wrep.at[s, nxt], wrepv.at[b], stsem)

                @pl.when(cur >= 2)
                def _wait_dst():
                    pltpu.make_async_copy(dst3.at[s, cur], dstv.at[b],
                                          dsem).wait()

                pltpu.async_copy(rows.at[b], accum.at[dstv.at[b]], scsem,
                                 add=True)

                @pl.when(nxt < NCH)
                def _launch_next():
                    pltpu.make_async_copy(srcT.at[t, s, nxt], srcv.at[b],
                                          stsem).wait()
                    pltpu.make_async_copy(wrep.at[s, nxt], wrepv.at[b],
                                          stsem).wait()
                    pltpu.make_async_copy(rows.at[b], accum.at[dstv.at[b]],
                                          scsem).wait()
                    pltpu.async_copy(dst3.at[s, nxt], dstv.at[b], dsem)
                    pltpu.async_copy(hw.at[srcv.at[b]], rows.at[b], gsems[b])

                @pl.when(nxt >= NCH)
                def _drain_scatter():
                    pltpu.make_async_copy(rows.at[b], accum.at[dstv.at[b]],
                                          scsem).wait()

        plsc.subcore_barrier()

        pltpu.sync_copy(accum.at[pl.ds(nbase, NPT)],
                        out.at[pl.ds(pl.multiple_of(t * N, 8) + nbase, NPT)])

        @pl.when(s == 0)
        def _wb_rem():
            pltpu.sync_copy(
                accum.at[pl.ds(NTILES * NPT, NREM)],
                out.at[pl.ds(pl.multiple_of(t * N + NTILES * NPT, 8), NREM)])


@functools.cache
def _make_spmm():
    return pl.kernel(
        _spmm_body,
        out_type=jax.ShapeDtypeStruct((TN, D), jnp.float32),
        mesh=plsc.VectorSubcoreMesh(core_axis_name="c", subcore_axis_name="s"),
        scratch_types=[
            pltpu.VMEM_SHARED((N, D), jnp.float32),
            pltpu.VMEM((2, CHUNK), jnp.int32),
            pltpu.VMEM((2, CHUNK), jnp.int32),
            pltpu.VMEM((2, CHUNK, 16), jnp.float32),
            pltpu.VMEM((2, CHUNK, D), jnp.float32),
            pltpu.SemaphoreType.DMA,
            pltpu.SemaphoreType.DMA,
            pltpu.SemaphoreType.DMA,
            pltpu.SemaphoreType.DMA,
            pltpu.SemaphoreType.DMA,
        ],
    )


def _spmm(hw, srcT, dst3, wrep):
    return _make_spmm()(hw, srcT, dst3, wrep)


BR = 4000


def _mm_body(x_ref, w_ref, b_ref, o_ref):
    o_ref[...] = (jnp.dot(x_ref[...], w_ref[...],
                          preferred_element_type=jnp.float32) + b_ref[...])


def _mm(x, w, b):
    return pl.pallas_call(
        _mm_body,
        grid=(TN // BR,),
        in_specs=[
            pl.BlockSpec((BR, D), lambda i: (i, 0)),
            pl.BlockSpec((D, D), lambda i: (0, 0)),
            pl.BlockSpec((1, D), lambda i: (0, 0)),
        ],
        out_specs=pl.BlockSpec((BR, D), lambda i: (i, 0)),
        out_shape=jax.ShapeDtypeStruct((TN, D), jnp.float32),
    )(x, w, b.reshape(1, D))


def _stats_body(x_ref, o_ref):
    i = pl.program_id(0)

    @pl.when(i == 0)
    def _():
        o_ref[...] = jnp.zeros_like(o_ref)

    x = x_ref[...]
    s = jnp.sum(x, axis=0, keepdims=True)
    ss = jnp.sum(x * x, axis=0, keepdims=True)
    o_ref[...] += jnp.concatenate([s, ss], axis=0)


def _stats(x):
    return pl.pallas_call(
        _stats_body,
        grid=(TN // BR,),
        in_specs=[pl.BlockSpec((BR, D), lambda i: (i, 0))],
        out_specs=pl.BlockSpec((2, D), lambda i: (0, 0)),
        out_shape=jax.ShapeDtypeStruct((2, D), jnp.float32),
    )(x)


def _bnmm_body(x_ref, sc_ref, sh_ref, w_ref, b_ref, o_ref):
    y = jnp.maximum(x_ref[...] * sc_ref[...] + sh_ref[...], 0.0)
    o_ref[...] = (jnp.dot(y, w_ref[...], preferred_element_type=jnp.float32)
                  + b_ref[...])


def _bnmm(x, scale, shift, w, b):
    return pl.pallas_call(
        _bnmm_body,
        grid=(TN // BR,),
        in_specs=[
            pl.BlockSpec((BR, D), lambda i: (i, 0)),
            pl.BlockSpec((1, D), lambda i: (0, 0)),
            pl.BlockSpec((1, D), lambda i: (0, 0)),
            pl.BlockSpec((D, D), lambda i: (0, 0)),
            pl.BlockSpec((1, D), lambda i: (0, 0)),
        ],
        out_specs=pl.BlockSpec((BR, D), lambda i: (i, 0)),
        out_shape=jax.ShapeDtypeStruct((TN, D), jnp.float32),
    )(x, scale.reshape(1, D), shift.reshape(1, D), w, b.reshape(1, D))


BNF = 400


def _final_body(x_ref, sc_ref, sh_ref, wq_ref, wk_ref, wv_ref, wa_ref,
                emb_ref, al_ref, wf_ref, bf_ref, o_ref):
    scale = sc_ref[...].reshape(1, 1, D)
    shift = sh_ref[...].reshape(1, 1, D)
    y = jnp.maximum(x_ref[...] * scale + shift, 0.0)
    y2 = y.reshape(T * BNF, D)

    q = jnp.dot(y2, wq_ref[...], preferred_element_type=jnp.float32)
    k = jnp.dot(y2, wk_ref[...], preferred_element_type=jnp.float32)
    v = jnp.dot(y2, wv_ref[...], preferred_element_type=jnp.float32)
    q = q.reshape(T, BNF, D)
    k = k.reshape(T, BNF, D)
    v = v.reshape(T, BNF, D)

    m4 = (lax.broadcasted_iota(jnp.int32, (D, H), 0) // DQ
          == lax.broadcasted_iota(jnp.int32, (D, H), 1)).astype(jnp.float32)
    m4t = m4.T
    inv = 1.0 / (DQ ** 0.5)

    outs = []
    for ti in range(T):
        svals = []
        for tj in range(ti + 1):
            p = q[ti] * k[tj]
            svals.append(jnp.dot(p, m4,
                                 preferred_element_type=jnp.float32) * inv)
        m = svals[0]
        for tj in range(1, ti + 1):
            m = jnp.maximum(m, svals[tj])
        es = [jnp.exp(sv - m) for sv in svals]
        denom = es[0]
        for tj in range(1, ti + 1):
            denom = denom + es[tj]
        acc = jnp.zeros((BNF, D), jnp.float32)
        for tj in range(ti + 1):
            a = jnp.dot(es[tj] / denom, m4t,
                        preferred_element_type=jnp.float32)
            acc = acc + a * v[tj]
        outs.append(acc)

    aggf = jnp.zeros((BNF, D), jnp.float32)
    for ti in range(T):
        wgt = jnp.dot(outs[ti], wa_ref[...],
                      preferred_element_type=jnp.float32)
        aggf = aggf + wgt * outs[ti]

    a_sig = jax.nn.sigmoid(al_ref[...])
    embed = (1.0 - a_sig) * emb_ref[...] + a_sig * aggf
    o_ref[...] = (jnp.dot(embed, wf_ref[...],
                          preferred_element_type=jnp.float32) + bf_ref[...])


def _final(x3, scale, shift, wq, wk, wv, wa, emb, alpha, wf, bf):
    return pl.pallas_call(
        _final_body,
        grid=(N // BNF,),
        in_specs=[
            pl.BlockSpec((T, BNF, D), lambda i: (0, i, 0)),
            pl.BlockSpec((1, D), lambda i: (0, 0)),
            pl.BlockSpec((1, D), lambda i: (0, 0)),
            pl.BlockSpec((D, D), lambda i: (0, 0)),
            pl.BlockSpec((D, D), lambda i: (0, 0)),
            pl.BlockSpec((D, D), lambda i: (0, 0)),
            pl.BlockSpec((D, 1), lambda i: (0, 0)),
            pl.BlockSpec((BNF, D), lambda i: (i, 0)),
            pl.BlockSpec((BNF, 1), lambda i: (i, 0)),
            pl.BlockSpec((D, 1), lambda i: (0, 0)),
            pl.BlockSpec((1, 1), lambda i: (0, 0)),
        ],
        out_specs=pl.BlockSpec((BNF, 1), lambda i: (i, 0)),
        out_shape=jax.ShapeDtypeStruct((N, 1), jnp.float32),
    )(x3, scale.reshape(1, D), shift.reshape(1, D), wq, wk, wv, wa,
      emb, alpha, wf, bf.reshape(1, 1))


def kernel(node_features, edge_index, edges_weight, lengths, nodes,
           users_frequency, W_gcn1, b_gcn1, gamma1, beta1, W_gcn2, b_gcn2,
           gamma2, beta2, Wq_att, Wk_att, Wv_att, Wq_agg, emb_table, alpha,
           W_fc, b_fc):
    x = node_features.transpose(1, 0, 2).reshape(TN, D)

    src = edge_index[0]
    dst = edge_index[1]
    pad = EPAD - E
    zpad = jnp.zeros((pad,), jnp.int32)
    src3 = jnp.concatenate([src, zpad]).reshape(NTILES, NCH, CHUNK)
    dst3 = jnp.concatenate([dst, zpad]).reshape(NTILES, NCH, CHUNK)
    wp = jnp.concatenate([edges_weight, jnp.zeros((pad,), jnp.float32)])
    wrep = jnp.broadcast_to(wp[:, None], (EPAD, 16)
                            ).reshape(NTILES, NCH, CHUNK, 16)
    toff = (jnp.arange(T, dtype=jnp.int32) * N)[:, None, None, None]
    srcT = src3[None] + toff

    ntf = jnp.float32(TN)

    hw1 = _mm(x, W_gcn1, b_gcn1)
    s1 = _spmm(hw1, srcT, dst3, wrep)
    st1 = _stats(s1)
    mean1 = st1[0] / ntf
    var1 = st1[1] / ntf - mean1 * mean1
    sc1 = gamma1 * lax.rsqrt(var1 + 1e-5)
    sh1 = beta1 - mean1 * sc1

    hw2 = _bnmm(s1, sc1, sh1, W_gcn2, b_gcn2)
    s2 = _spmm(hw2, srcT, dst3, wrep)
    st2 = _stats(s2)
    mean2 = st2[0] / ntf
    var2 = st2[1] / ntf - mean2 * mean2
    sc2 = gamma2 * lax.rsqrt(var2 + 1e-5)
    sh2 = beta2 - mean2 * sc2

    out = _final(s2.reshape(T, N, D), sc2, sh2, Wq_att, Wk_att, Wv_att,
                 Wq_agg, emb_table, alpha.reshape(N, 1), W_fc, b_fc)
    return out.reshape(N)


# speedup vs baseline: 1.0008x; 1.0008x over previous
import functools

import jax
import jax.numpy as jnp
from jax import lax
from jax.experimental import pallas as pl
from jax.experimental.pallas import tpu as pltpu
from jax.experimental.pallas import tpu_sc as plsc

N = 10000
T = 8
D = 128
H = 4
DQ = 32
E = 160000

NTILES = 16
NCORES = 2
CHUNK = 72
NCH = 140
EPT = NCH * CHUNK
EPAD = EPT * NTILES
NPT = 624
NREM = N - NTILES * NPT
TPC = T // NCORES
IWCH = ((0, 128), (128, 128), (256, 128), (384, 128), (512, 112))

TN = T * N


def _spmm_body(hw, srcT, dst3, wrep, out, accum, srcv, dstv, wrepv, rows,
               gsem0, gsem1, stsem, dsem):
    c = lax.axis_index("c")
    s = lax.axis_index("s")
    gsems = (gsem0, gsem1)

    for jj in range(TPC):
        t = c * TPC + jj

        for b in range(2):
            pltpu.async_copy(srcT.at[t, s, b], srcv.at[b], stsem)
            pltpu.async_copy(dst3.at[s, b], dstv.at[b], stsem)
            pltpu.async_copy(wrep.at[s, b], wrepv.at[b], stsem)
        for b in range(2):
            pltpu.make_async_copy(srcT.at[t, s, b], srcv.at[b], stsem).wait()
            pltpu.make_async_copy(dst3.at[s, b], dstv.at[b], stsem).wait()
            pltpu.make_async_copy(wrep.at[s, b], wrepv.at[b], stsem).wait()
            pltpu.async_copy(hw.at[srcv.at[b]], rows.at[b], gsems[b])

        nbase = pl.multiple_of(s * NPT, 8)
        pltpu.sync_copy(hw.at[pl.ds(pl.multiple_of(t * N, 8) + nbase, NPT)],
                        accum.at[pl.ds(nbase, NPT)])

        @pl.when(s == 0)
        def _init_rem():
            pltpu.sync_copy(
                hw.at[pl.ds(pl.multiple_of(t * N + NTILES * NPT, 8), NREM)],
                accum.at[pl.ds(NTILES * NPT, NREM)])

        plsc.subcore_barrier()

        @pl.loop(0, NCH, step=2)
        def _chunk2(ch):
            for b in range(2):
                cur = ch + b
                nxt = cur + 2
                pltpu.make_async_copy(hw.at[srcv.at[b]], rows.at[b],
                                      gsems[b]).wait()

                @pl.loop(0, CHUNK)
                def _edge(i):
                    wspl = wrepv[b, i]
                    for k in range(D // 16):
                        sl = pl.ds(k * 16, 16)
                        rows[b, i, sl] = rows[b, i, sl] * wspl

                @pl.when(nxt < NCH)
                def _stage_next():
                    pltpu.async_copy(srcT.at[t, s, nxt], srcv.at[b], stsem)
                    pltpu.async_copy(wrep.at[s, nxt], wrepv.at[b], stsem)

                @pl.when(cur >= 2)
                def _wait_dst():
                    pltpu.make_async_copy(dst3.at[s, cur], dstv.at[b],
                                          dsem).wait()

                pltpu.sync_copy(rows.at[b], accum.at[dstv.at[b]], add=True)

                @pl.when(nxt < NCH)
                def _launch_next():
                    pltpu.make_async_copy(srcT.at[t, s, nxt], srcv.at[b],
                                          stsem).wait()
                    pltpu.make_async_copy(wrep.at[s, nxt], wrepv.at[b],
                                          stsem).wait()
                    pltpu.async_copy(dst3.at[s, nxt], dstv.at[b], dsem)
                    pltpu.async_copy(hw.at[srcv.at[b]], rows.at[b], gsems[b])

        plsc.subcore_barrier()

        pltpu.sync_copy(accum.at[pl.ds(nbase, NPT)],
                        out.at[pl.ds(pl.multiple_of(t * N, 8) + nbase, NPT)])

        @pl.when(s == 0)
        def _wb_rem():
            pltpu.sync_copy(
                accum.at[pl.ds(NTILES * NPT, NREM)],
                out.at[pl.ds(pl.multiple_of(t * N + NTILES * NPT, 8), NREM)])


@functools.cache
def _make_spmm():
    return pl.kernel(
        _spmm_body,
        out_type=jax.ShapeDtypeStruct((TN, D), jnp.float32),
        mesh=plsc.VectorSubcoreMesh(core_axis_name="c", subcore_axis_name="s"),
        scratch_types=[
            pltpu.VMEM_SHARED((N, D), jnp.float32),
            pltpu.VMEM((2, CHUNK), jnp.int32),
            pltpu.VMEM((2, CHUNK), jnp.int32),
            pltpu.VMEM((2, CHUNK, 16), jnp.float32),
            pltpu.VMEM((2, CHUNK, D), jnp.float32),
            pltpu.SemaphoreType.DMA,
            pltpu.SemaphoreType.DMA,
            pltpu.SemaphoreType.DMA,
            pltpu.SemaphoreType.DMA,
        ],
    )


def _spmm(hw, srcT, dst3, wrep):
    return _make_spmm()(hw, srcT, dst3, wrep)


BR = 4000


def _mm_body(x_ref, w_ref, b_ref, o_ref):
    o_ref[...] = (jnp.dot(x_ref[...], w_ref[...],
                          preferred_element_type=jnp.float32) + b_ref[...])


def _mm(x, w, b):
    return pl.pallas_call(
        _mm_body,
        grid=(TN // BR,),
        in_specs=[
            pl.BlockSpec((BR, D), lambda i: (i, 0)),
            pl.BlockSpec((D, D), lambda i: (0, 0)),
            pl.BlockSpec((1, D), lambda i: (0, 0)),
        ],
        out_specs=pl.BlockSpec((BR, D), lambda i: (i, 0)),
        out_shape=jax.ShapeDtypeStruct((TN, D), jnp.float32),
    )(x, w, b.reshape(1, D))


def _stats_body(x_ref, o_ref):
    i = pl.program_id(0)

    @pl.when(i == 0)
    def _():
        o_ref[...] = jnp.zeros_like(o_ref)

    x = x_ref[...]
    s = jnp.sum(x, axis=0, keepdims=True)
    ss = jnp.sum(x * x, axis=0, keepdims=True)
    o_ref[...] += jnp.concatenate([s, ss], axis=0)


def _stats(x):
    return pl.pallas_call(
        _stats_body,
        grid=(TN // BR,),
        in_specs=[pl.BlockSpec((BR, D), lambda i: (i, 0))],
        out_specs=pl.BlockSpec((2, D), lambda i: (0, 0)),
        out_shape=jax.ShapeDtypeStruct((2, D), jnp.float32),
    )(x)


def _bnmm_body(x_ref, sc_ref, sh_ref, w_ref, b_ref, o_ref):
    y = jnp.maximum(x_ref[...] * sc_ref[...] + sh_ref[...], 0.0)
    o_ref[...] = (jnp.dot(y, w_ref[...], preferred_element_type=jnp.float32)
                  + b_ref[...])


def _bnmm(x, scale, shift, w, b):
    return pl.pallas_call(
        _bnmm_body,
        grid=(TN // BR,),
        in_specs=[
            pl.BlockSpec((BR, D), lambda i: (i, 0)),
            pl.BlockSpec((1, D), lambda i: (0, 0)),
            pl.BlockSpec((1, D), lambda i: (0, 0)),
            pl.BlockSpec((D, D), lambda i: (0, 0)),
            pl.BlockSpec((1, D), lambda i: (0, 0)),
        ],
        out_specs=pl.BlockSpec((BR, D), lambda i: (i, 0)),
        out_shape=jax.ShapeDtypeStruct((TN, D), jnp.float32),
    )(x, scale.reshape(1, D), shift.reshape(1, D), w, b.reshape(1, D))


BNF = 400


def _final_body(x_ref, sc_ref, sh_ref, wq_ref, wk_ref, wv_ref, wa_ref,
                emb_ref, al_ref, wf_ref, bf_ref, o_ref):
    scale = sc_ref[...].reshape(1, 1, D)
    shift = sh_ref[...].reshape(1, 1, D)
    y = jnp.maximum(x_ref[...] * scale + shift, 0.0)
    y2 = y.reshape(T * BNF, D)

    q = jnp.dot(y2, wq_ref[...], preferred_element_type=jnp.float32)
    k = jnp.dot(y2, wk_ref[...], preferred_element_type=jnp.float32)
    v = jnp.dot(y2, wv_ref[...], preferred_element_type=jnp.float32)
    q = q.reshape(T, BNF, D)
    k = k.reshape(T, BNF, D)
    v = v.reshape(T, BNF, D)

    m4 = (lax.broadcasted_iota(jnp.int32, (D, H), 0) // DQ
          == lax.broadcasted_iota(jnp.int32, (D, H), 1)).astype(jnp.float32)
    m4t = m4.T
    inv = 1.0 / (DQ ** 0.5)

    outs = []
    for ti in range(T):
        svals = []
        for tj in range(ti + 1):
            p = q[ti] * k[tj]
            svals.append(jnp.dot(p, m4,
                                 preferred_element_type=jnp.float32) * inv)
        m = svals[0]
        for tj in range(1, ti + 1):
            m = jnp.maximum(m, svals[tj])
        es = [jnp.exp(sv - m) for sv in svals]
        denom = es[0]
        for tj in range(1, ti + 1):
            denom = denom + es[tj]
        acc = jnp.zeros((BNF, D), jnp.float32)
        for tj in range(ti + 1):
            a = jnp.dot(es[tj] / denom, m4t,
                        preferred_element_type=jnp.float32)
            acc = acc + a * v[tj]
        outs.append(acc)

    aggf = jnp.zeros((BNF, D), jnp.float32)
    for ti in range(T):
        wgt = jnp.dot(outs[ti], wa_ref[...],
                      preferred_element_type=jnp.float32)
        aggf = aggf + wgt * outs[ti]

    a_sig = jax.nn.sigmoid(al_ref[...])
    embed = (1.0 - a_sig) * emb_ref[...] + a_sig * aggf
    o_ref[...] = (jnp.dot(embed, wf_ref[...],
                          preferred_element_type=jnp.float32) + bf_ref[...])


def _final(x3, scale, shift, wq, wk, wv, wa, emb, alpha, wf, bf):
    return pl.pallas_call(
        _final_body,
        grid=(N // BNF,),
        in_specs=[
            pl.BlockSpec((T, BNF, D), lambda i: (0, i, 0)),
            pl.BlockSpec((1, D), lambda i: (0, 0)),
            pl.BlockSpec((1, D), lambda i: (0, 0)),
            pl.BlockSpec((D, D), lambda i: (0, 0)),
            pl.BlockSpec((D, D), lambda i: (0, 0)),
            pl.BlockSpec((D, D), lambda i: (0, 0)),
            pl.BlockSpec((D, 1), lambda i: (0, 0)),
            pl.BlockSpec((BNF, D), lambda i: (i, 0)),
            pl.BlockSpec((BNF, 1), lambda i: (i, 0)),
            pl.BlockSpec((D, 1), lambda i: (0, 0)),
            pl.BlockSpec((1, 1), lambda i: (0, 0)),
        ],
        out_specs=pl.BlockSpec((BNF, 1), lambda i: (i, 0)),
        out_shape=jax.ShapeDtypeStruct((N, 1), jnp.float32),
    )(x3, scale.reshape(1, D), shift.reshape(1, D), wq, wk, wv, wa,
      emb, alpha, wf, bf.reshape(1, 1))


def kernel(node_features, edge_index, edges_weight, lengths, nodes,
           users_frequency, W_gcn1, b_gcn1, gamma1, beta1, W_gcn2, b_gcn2,
           gamma2, beta2, Wq_att, Wk_att, Wv_att, Wq_agg, emb_table, alpha,
           W_fc, b_fc):
    x = node_features.transpose(1, 0, 2).reshape(TN, D)

    src = edge_index[0]
    dst = edge_index[1]
    pad = EPAD - E
    zpad = jnp.zeros((pad,), jnp.int32)
    src3 = jnp.concatenate([src, zpad]).reshape(NTILES, NCH, CHUNK)
    dst3 = jnp.concatenate([dst, zpad]).reshape(NTILES, NCH, CHUNK)
    wp = jnp.concatenate([edges_weight, jnp.zeros((pad,), jnp.float32)])
    wrep = jnp.broadcast_to(wp[:, None], (EPAD, 16)
                            ).reshape(NTILES, NCH, CHUNK, 16)
    toff = (jnp.arange(T, dtype=jnp.int32) * N)[:, None, None, None]
    srcT = src3[None] + toff

    ntf = jnp.float32(TN)

    hw1 = _mm(x, W_gcn1, b_gcn1)
    s1 = _spmm(hw1, srcT, dst3, wrep)
    st1 = _stats(s1)
    mean1 = st1[0] / ntf
    var1 = st1[1] / ntf - mean1 * mean1
    sc1 = gamma1 * lax.rsqrt(var1 + 1e-5)
    sh1 = beta1 - mean1 * sc1

    hw2 = _bnmm(s1, sc1, sh1, W_gcn2, b_gcn2)
    s2 = _spmm(hw2, srcT, dst3, wrep)
    st2 = _stats(s2)
    mean2 = st2[0] / ntf
    var2 = st2[1] / ntf - mean2 * mean2
    sc2 = gamma2 * lax.rsqrt(var2 + 1e-5)
    sh2 = beta2 - mean2 * sc2

    out = _final(s2.reshape(T, N, D), sc2, sh2, Wq_att, Wk_att, Wv_att,
                 Wq_agg, emb_table, alpha.reshape(N, 1), W_fc, b_fc)
    return out.reshape(N)


# speedup vs baseline: 1.1313x; 1.1303x over previous
import functools

import jax
import jax.numpy as jnp
from jax import lax
from jax.experimental import pallas as pl
from jax.experimental.pallas import tpu as pltpu
from jax.experimental.pallas import tpu_sc as plsc

N = 10000
T = 8
D = 128
H = 4
DQ = 32
E = 160000

NTILES = 16
NCORES = 2
CHUNK = 76
NCH = 132
EPT = NCH * CHUNK
EPAD = EPT * NTILES
NPT = 624
NREM = N - NTILES * NPT
TPC = T // NCORES
IWCH = ((0, 128), (128, 128), (256, 128), (384, 128), (512, 112))

TN = T * N


def _spmm_body(hw, srcT, dst3, wrep, out, accum, srcv, dstv, wrepv, rows,
               gsem0, gsem1, stsem, dsem):
    c = lax.axis_index("c")
    s = lax.axis_index("s")
    gsems = (gsem0, gsem1)

    for jj in range(TPC):
        t = c * TPC + jj

        for b in range(2):
            pltpu.async_copy(srcT.at[t, s, b], srcv.at[b], stsem)
            pltpu.async_copy(dst3.at[s, b], dstv.at[b], stsem)
            pltpu.async_copy(wrep.at[s, b], wrepv.at[b], stsem)
        for b in range(2):
            pltpu.make_async_copy(srcT.at[t, s, b], srcv.at[b], stsem).wait()
            pltpu.make_async_copy(dst3.at[s, b], dstv.at[b], stsem).wait()
            pltpu.make_async_copy(wrep.at[s, b], wrepv.at[b], stsem).wait()
            pltpu.async_copy(hw.at[srcv.at[b]], rows.at[b], gsems[b])

        nbase = pl.multiple_of(s * NPT, 8)
        pltpu.sync_copy(hw.at[pl.ds(pl.multiple_of(t * N, 8) + nbase, NPT)],
                        accum.at[pl.ds(nbase, NPT)])

        @pl.when(s == 0)
        def _init_rem():
            pltpu.sync_copy(
                hw.at[pl.ds(pl.multiple_of(t * N + NTILES * NPT, 8), NREM)],
                accum.at[pl.ds(NTILES * NPT, NREM)])

        plsc.subcore_barrier()

        @pl.loop(0, NCH, step=2)
        def _chunk2(ch):
            for b in range(2):
                cur = ch + b
                nxt = cur + 2
                pltpu.make_async_copy(hw.at[srcv.at[b]], rows.at[b],
                                      gsems[b]).wait()

                @pl.loop(0, CHUNK)
                def _edge(i):
                    wspl = wrepv[b, i]
                    for k in range(D // 16):
                        sl = pl.ds(k * 16, 16)
                        rows[b, i, sl] = rows[b, i, sl] * wspl

                @pl.when(nxt < NCH)
                def _stage_next():
                    pltpu.async_copy(srcT.at[t, s, nxt], srcv.at[b], stsem)
                    pltpu.async_copy(wrep.at[s, nxt], wrepv.at[b], stsem)

                @pl.when(cur >= 2)
                def _wait_dst():
                    pltpu.make_async_copy(dst3.at[s, cur], dstv.at[b],
                                          dsem).wait()

                pltpu.sync_copy(rows.at[b], accum.at[dstv.at[b]], add=True)

                @pl.when(nxt < NCH)
                def _launch_next():
                    pltpu.make_async_copy(srcT.at[t, s, nxt], srcv.at[b],
                                          stsem).wait()
                    pltpu.make_async_copy(wrep.at[s, nxt], wrepv.at[b],
                                          stsem).wait()
                    pltpu.async_copy(dst3.at[s, nxt], dstv.at[b], dsem)
                    pltpu.async_copy(hw.at[srcv.at[b]], rows.at[b], gsems[b])

        plsc.subcore_barrier()

        pltpu.sync_copy(accum.at[pl.ds(nbase, NPT)],
                        out.at[pl.ds(pl.multiple_of(t * N, 8) + nbase, NPT)])

        @pl.when(s == 0)
        def _wb_rem():
            pltpu.sync_copy(
                accum.at[pl.ds(NTILES * NPT, NREM)],
                out.at[pl.ds(pl.multiple_of(t * N + NTILES * NPT, 8), NREM)])


@functools.cache
def _make_spmm():
    return pl.kernel(
        _spmm_body,
        out_type=jax.ShapeDtypeStruct((TN, D), jnp.float32),
        mesh=plsc.VectorSubcoreMesh(core_axis_name="c", subcore_axis_name="s"),
        scratch_types=[
            pltpu.VMEM_SHARED((N, D), jnp.float32),
            pltpu.VMEM((2, CHUNK), jnp.int32),
            pltpu.VMEM((2, CHUNK), jnp.int32),
            pltpu.VMEM((2, CHUNK, 16), jnp.float32),
            pltpu.VMEM((2, CHUNK, D), jnp.float32),
            pltpu.SemaphoreType.DMA,
            pltpu.SemaphoreType.DMA,
            pltpu.SemaphoreType.DMA,
            pltpu.SemaphoreType.DMA,
        ],
    )


def _spmm(hw, srcT, dst3, wrep):
    return _make_spmm()(hw, srcT, dst3, wrep)


BR = 4000


def _mm_body(x_ref, w_ref, b_ref, o_ref):
    o_ref[...] = (jnp.dot(x_ref[...], w_ref[...],
                          preferred_element_type=jnp.float32) + b_ref[...])


def _mm(x, w, b):
    return pl.pallas_call(
        _mm_body,
        grid=(TN // BR,),
        in_specs=[
            pl.BlockSpec((BR, D), lambda i: (i, 0)),
            pl.BlockSpec((D, D), lambda i: (0, 0)),
            pl.BlockSpec((1, D), lambda i: (0, 0)),
        ],
        out_specs=pl.BlockSpec((BR, D), lambda i: (i, 0)),
        out_shape=jax.ShapeDtypeStruct((TN, D), jnp.float32),
    )(x, w, b.reshape(1, D))


def _stats_body(x_ref, o_ref):
    i = pl.program_id(0)

    @pl.when(i == 0)
    def _():
        o_ref[...] = jnp.zeros_like(o_ref)

    x = x_ref[...]
    s = jnp.sum(x, axis=0, keepdims=True)
    ss = jnp.sum(x * x, axis=0, keepdims=True)
    o_ref[...] += jnp.concatenate([s, ss], axis=0)


def _stats(x):
    return pl.pallas_call(
        _stats_body,
        grid=(TN // BR,),
        in_specs=[pl.BlockSpec((BR, D), lambda i: (i, 0))],
        out_specs=pl.BlockSpec((2, D), lambda i: (0, 0)),
        out_shape=jax.ShapeDtypeStruct((2, D), jnp.float32),
    )(x)


def _bnmm_body(x_ref, sc_ref, sh_ref, w_ref, b_ref, o_ref):
    y = jnp.maximum(x_ref[...] * sc_ref[...] + sh_ref[...], 0.0)
    o_ref[...] = (jnp.dot(y, w_ref[...], preferred_element_type=jnp.float32)
                  + b_ref[...])


def _bnmm(x, scale, shift, w, b):
    return pl.pallas_call(
        _bnmm_body,
        grid=(TN // BR,),
        in_specs=[
            pl.BlockSpec((BR, D), lambda i: (i, 0)),
            pl.BlockSpec((1, D), lambda i: (0, 0)),
            pl.BlockSpec((1, D), lambda i: (0, 0)),
            pl.BlockSpec((D, D), lambda i: (0, 0)),
            pl.BlockSpec((1, D), lambda i: (0, 0)),
        ],
        out_specs=pl.BlockSpec((BR, D), lambda i: (i, 0)),
        out_shape=jax.ShapeDtypeStruct((TN, D), jnp.float32),
    )(x, scale.reshape(1, D), shift.reshape(1, D), w, b.reshape(1, D))


BNF = 400


def _final_body(x_ref, sc_ref, sh_ref, wq_ref, wk_ref, wv_ref, wa_ref,
                emb_ref, al_ref, wf_ref, bf_ref, o_ref):
    scale = sc_ref[...].reshape(1, 1, D)
    shift = sh_ref[...].reshape(1, 1, D)
    y = jnp.maximum(x_ref[...] * scale + shift, 0.0)
    y2 = y.reshape(T * BNF, D)

    q = jnp.dot(y2, wq_ref[...], preferred_element_type=jnp.float32)
    k = jnp.dot(y2, wk_ref[...], preferred_element_type=jnp.float32)
    v = jnp.dot(y2, wv_ref[...], preferred_element_type=jnp.float32)
    q = q.reshape(T, BNF, D)
    k = k.reshape(T, BNF, D)
    v = v.reshape(T, BNF, D)

    m4 = (lax.broadcasted_iota(jnp.int32, (D, H), 0) // DQ
          == lax.broadcasted_iota(jnp.int32, (D, H), 1)).astype(jnp.float32)
    m4t = m4.T
    inv = 1.0 / (DQ ** 0.5)

    outs = []
    for ti in range(T):
        svals = []
        for tj in range(ti + 1):
            p = q[ti] * k[tj]
            svals.append(jnp.dot(p, m4,
                                 preferred_element_type=jnp.float32) * inv)
        m = svals[0]
        for tj in range(1, ti + 1):
            m = jnp.maximum(m, svals[tj])
        es = [jnp.exp(sv - m) for sv in svals]
        denom = es[0]
        for tj in range(1, ti + 1):
            denom = denom + es[tj]
        acc = jnp.zeros((BNF, D), jnp.float32)
        for tj in range(ti + 1):
            a = jnp.dot(es[tj] / denom, m4t,
                        preferred_element_type=jnp.float32)
            acc = acc + a * v[tj]
        outs.append(acc)

    aggf = jnp.zeros((BNF, D), jnp.float32)
    for ti in range(T):
        wgt = jnp.dot(outs[ti], wa_ref[...],
                      preferred_element_type=jnp.float32)
        aggf = aggf + wgt * outs[ti]

    a_sig = jax.nn.sigmoid(al_ref[...])
    embed = (1.0 - a_sig) * emb_ref[...] + a_sig * aggf
    o_ref[...] = (jnp.dot(embed, wf_ref[...],
                          preferred_element_type=jnp.float32) + bf_ref[...])


def _final(x3, scale, shift, wq, wk, wv, wa, emb, alpha, wf, bf):
    return pl.pallas_call(
        _final_body,
        grid=(N // BNF,),
        in_specs=[
            pl.BlockSpec((T, BNF, D), lambda i: (0, i, 0)),
            pl.BlockSpec((1, D), lambda i: (0, 0)),
            pl.BlockSpec((1, D), lambda i: (0, 0)),
            pl.BlockSpec((D, D), lambda i: (0, 0)),
            pl.BlockSpec((D, D), lambda i: (0, 0)),
            pl.BlockSpec((D, D), lambda i: (0, 0)),
            pl.BlockSpec((D, 1), lambda i: (0, 0)),
            pl.BlockSpec((BNF, D), lambda i: (i, 0)),
            pl.BlockSpec((BNF, 1), lambda i: (i, 0)),
            pl.BlockSpec((D, 1), lambda i: (0, 0)),
            pl.BlockSpec((1, 1), lambda i: (0, 0)),
        ],
        out_specs=pl.BlockSpec((BNF, 1), lambda i: (i, 0)),
        out_shape=jax.ShapeDtypeStruct((N, 1), jnp.float32),
    )(x3, scale.reshape(1, D), shift.reshape(1, D), wq, wk, wv, wa,
      emb, alpha, wf, bf.reshape(1, 1))


def kernel(node_features, edge_index, edges_weight, lengths, nodes,
           users_frequency, W_gcn1, b_gcn1, gamma1, beta1, W_gcn2, b_gcn2,
           gamma2, beta2, Wq_att, Wk_att, Wv_att, Wq_agg, emb_table, alpha,
           W_fc, b_fc):
    x = node_features.transpose(1, 0, 2).reshape(TN, D)

    src = edge_index[0]
    dst = edge_index[1]
    pad = EPAD - E
    zpad = jnp.zeros((pad,), jnp.int32)
    src3 = jnp.concatenate([src, zpad]).reshape(NTILES, NCH, CHUNK)
    dst3 = jnp.concatenate([dst, zpad]).reshape(NTILES, NCH, CHUNK)
    wp = jnp.concatenate([edges_weight, jnp.zeros((pad,), jnp.float32)])
    wrep = jnp.broadcast_to(wp[:, None], (EPAD, 16)
                            ).reshape(NTILES, NCH, CHUNK, 16)
    toff = (jnp.arange(T, dtype=jnp.int32) * N)[:, None, None, None]
    srcT = src3[None] + toff

    ntf = jnp.float32(TN)

    hw1 = _mm(x, W_gcn1, b_gcn1)
    s1 = _spmm(hw1, srcT, dst3, wrep)
    st1 = _stats(s1)
    mean1 = st1[0] / ntf
    var1 = st1[1] / ntf - mean1 * mean1
    sc1 = gamma1 * lax.rsqrt(var1 + 1e-5)
    sh1 = beta1 - mean1 * sc1

    hw2 = _bnmm(s1, sc1, sh1, W_gcn2, b_gcn2)
    s2 = _spmm(hw2, srcT, dst3, wrep)
    st2 = _stats(s2)
    mean2 = st2[0] / ntf
    var2 = st2[1] / ntf - mean2 * mean2
    sc2 = gamma2 * lax.rsqrt(var2 + 1e-5)
    sh2 = beta2 - mean2 * sc2

    out = _final(s2.reshape(T, N, D), sc2, sh2, Wq_att, Wk_att, Wv_att,
                 Wq_agg, emb_table, alpha.reshape(N, 1), W_fc, b_fc)
    return out.reshape(N)
